# baseline (speedup 1.0000x reference)
import jax
import jax.numpy as jnp
from jax.experimental import pallas as pl
from jax.experimental.pallas import tpu as pltpu

_LANE = 128
_SUBLANE = 8


def _round_up(x, m):
    return (x + m - 1) // m * m


def _pad2(w, rows, cols, dtype):
    return jnp.pad(w, ((0, rows - w.shape[0]), (0, cols - w.shape[1]))).astype(dtype)


def _fused_kernel(p_ref, f_ref, w1p_ref, w1f_ref, w2_ref, b2_ref,
                  w3l_ref, b3_ref, w3q_ref, w4_ref, b4_ref,
                  o_ref, bias_ref):
    t = pl.program_id(1)

    def points8():
        p = p_ref[0]
        p8 = jnp.pad(p, ((0, 0), (0, 5)))
        ones = jax.lax.broadcasted_iota(jnp.int32, p8.shape, 1) == 3
        return jnp.where(ones, 1.0, p8).astype(jnp.bfloat16)

    @pl.when(t == 0)
    def _encode():
        p8 = points8()
        fb = f_ref[0].astype(jnp.bfloat16)
        s = (jnp.dot(p8, w1p_ref[...], preferred_element_type=jnp.float32)
             + jnp.dot(fb, w1f_ref[...], preferred_element_type=jnp.float32))
        h = jnp.maximum(s.astype(jnp.bfloat16), 0)
        z = jnp.dot(h, w2_ref[...], preferred_element_type=jnp.float32)
        zm = jnp.max(z, axis=0, keepdims=True)
        lat = jnp.maximum(zm + b2_ref[...], 0.0).astype(jnp.bfloat16)
        bias_ref[...] = (jnp.dot(lat, w3l_ref[...],
                                 preferred_element_type=jnp.float32)
                         + b3_ref[...])

    @pl.when(t == 1)
    def _decode():
        p8 = points8()
        s = (jnp.dot(p8, w3q_ref[...], preferred_element_type=jnp.float32)
             + bias_ref[...])
        h = jnp.maximum(s.astype(jnp.bfloat16), 0)
        o_ref[0] = (jnp.dot(h, w4_ref[...], preferred_element_type=jnp.float32)
                    + b4_ref[...])


def kernel(input_points, input_features, w1p, w1f, b1, w2, b2,
           w3q, w3l, b3, w4, b4):
    B, N, _ = input_points.shape
    D = input_features.shape[-1]
    H = w1p.shape[-1]
    L = w2.shape[-1]
    Q = w4.shape[-1]

    D_pad = _round_up(D, 2 * _SUBLANE)
    H_pad = _round_up(H, _LANE)
    L_pad = _round_up(L, _LANE)
    Q_pad = _round_up(Q, _LANE)

    bf16 = jnp.bfloat16
    w1p8 = jnp.concatenate(
        [w1p, b1, jnp.zeros((4, H), w1p.dtype)], axis=0)
    w1p8 = _pad2(w1p8, 8, H_pad, bf16)
    w1fp = _pad2(w1f, D_pad, H_pad, bf16)
    w2p = _pad2(w2, H_pad, L_pad, bf16)
    b2f = _pad2(b2, 1, L_pad, jnp.float32)
    w3lp = _pad2(w3l, L_pad, H_pad, bf16)
    b3f = _pad2(b3, 1, H_pad, jnp.float32)
    w3q8 = _pad2(w3q, 8, H_pad, bf16)
    w4p = _pad2(w4, H_pad, Q_pad, bf16)
    b4f = _pad2(b4, 1, Q_pad, jnp.float32)

    feats = input_features
    if D_pad != D:
        feats = jnp.pad(feats, ((0, 0), (0, 0), (0, D_pad - D)))

    full = lambda shape: pl.BlockSpec(shape, lambda b, t: (0,) * len(shape))
    out_pad = pl.pallas_call(
        _fused_kernel,
        out_shape=jax.ShapeDtypeStruct((B, N, Q_pad), jnp.float32),
        grid=(B, 2),
        in_specs=[
            pl.BlockSpec((1, N, 3), lambda b, t: (b, 0, 0)),
            pl.BlockSpec((1, N, D_pad), lambda b, t: (b, 0, 0)),
            full((8, H_pad)),
            full((D_pad, H_pad)),
            full((H_pad, L_pad)),
            full((1, L_pad)),
            full((L_pad, H_pad)),
            full((1, H_pad)),
            full((8, H_pad)),
            full((H_pad, Q_pad)),
            full((1, Q_pad)),
        ],
        out_specs=pl.BlockSpec((1, N, Q_pad), lambda b, t: (b, 0, 0)),
        scratch_shapes=[pltpu.VMEM((1, H_pad), jnp.float32)],
        compiler_params=pltpu.CompilerParams(
            dimension_semantics=("parallel", "arbitrary")),
    )(input_points, feats, w1p8, w1fp, w2p, b2f, w3lp, b3f, w3q8, w4p, b4f)

    if Q_pad != Q:
        return out_pad[:, :, :Q]
    return out_pad


# speedup vs baseline: 1.0036x; 1.0036x over previous
import jax
import jax.numpy as jnp
from jax.experimental import pallas as pl
from jax.experimental.pallas import tpu as pltpu

_LANE = 128
_SUBLANE = 8


def _round_up(x, m):
    return (x + m - 1) // m * m


def _pad2(w, rows, cols, dtype):
    return jnp.pad(w, ((0, rows - w.shape[0]), (0, cols - w.shape[1]))).astype(dtype)


def _fused_kernel(p_ref, f_ref, w1p_ref, w1f_ref, w2_ref, b2_ref,
                  w3l_ref, b3_ref, w3q_ref, w4_ref, b4_ref, o_ref):
    p = p_ref[0]
    p8 = jnp.pad(p, ((0, 0), (0, 5)))
    ones = jax.lax.broadcasted_iota(jnp.int32, p8.shape, 1) == 3
    p8 = jnp.where(ones, 1.0, p8).astype(jnp.bfloat16)

    fb = f_ref[0].astype(jnp.bfloat16)
    s = (jnp.dot(p8, w1p_ref[...], preferred_element_type=jnp.float32)
         + jnp.dot(fb, w1f_ref[...], preferred_element_type=jnp.float32))
    h = jnp.maximum(s.astype(jnp.bfloat16), 0)
    z = jnp.dot(h, w2_ref[...], preferred_element_type=jnp.float32)
    zm = jnp.max(z, axis=0, keepdims=True)
    lat = jnp.maximum(zm + b2_ref[...], 0.0).astype(jnp.bfloat16)
    bias = (jnp.dot(lat, w3l_ref[...], preferred_element_type=jnp.float32)
            + b3_ref[...])

    s2 = (jnp.dot(p8, w3q_ref[...], preferred_element_type=jnp.float32)
          + bias)
    h2 = jnp.maximum(s2.astype(jnp.bfloat16), 0)
    o_ref[0] = (jnp.dot(h2, w4_ref[...], preferred_element_type=jnp.float32)
                + b4_ref[...])


def kernel(input_points, input_features, w1p, w1f, b1, w2, b2,
           w3q, w3l, b3, w4, b4):
    B, N, _ = input_points.shape
    D = input_features.shape[-1]
    H = w1p.shape[-1]
    L = w2.shape[-1]
    Q = w4.shape[-1]

    D_pad = _round_up(D, 2 * _SUBLANE)
    H_pad = _round_up(H, _LANE)
    L_pad = _round_up(L, _LANE)
    Q_pad = _round_up(Q, _LANE)

    bf16 = jnp.bfloat16
    w1p8 = jnp.concatenate(
        [w1p, b1, jnp.zeros((4, H), w1p.dtype)], axis=0)
    w1p8 = _pad2(w1p8, 8, H_pad, bf16)
    w1fp = _pad2(w1f, D_pad, H_pad, bf16)
    w2p = _pad2(w2, H_pad, L_pad, bf16)
    b2f = _pad2(b2, 1, L_pad, jnp.float32)
    w3lp = _pad2(w3l, L_pad, H_pad, bf16)
    b3f = _pad2(b3, 1, H_pad, jnp.float32)
    w3q8 = _pad2(w3q, 8, H_pad, bf16)
    w4p = _pad2(w4, H_pad, Q_pad, bf16)
    b4f = _pad2(b4, 1, Q_pad, jnp.float32)

    feats = input_features
    if D_pad != D:
        feats = jnp.pad(feats, ((0, 0), (0, 0), (0, D_pad - D)))

    full = lambda shape: pl.BlockSpec(shape, lambda b: (0,) * len(shape))
    out_pad = pl.pallas_call(
        _fused_kernel,
        out_shape=jax.ShapeDtypeStruct((B, N, Q_pad), jnp.float32),
        grid=(B,),
        in_specs=[
            pl.BlockSpec((1, N, 3), lambda b: (b, 0, 0)),
            pl.BlockSpec((1, N, D_pad), lambda b: (b, 0, 0)),
            full((8, H_pad)),
            full((D_pad, H_pad)),
            full((H_pad, L_pad)),
            full((1, L_pad)),
            full((L_pad, H_pad)),
            full((1, H_pad)),
            full((8, H_pad)),
            full((H_pad, Q_pad)),
            full((1, Q_pad)),
        ],
        out_specs=pl.BlockSpec((1, N, Q_pad), lambda b: (b, 0, 0)),
        compiler_params=pltpu.CompilerParams(
            dimension_semantics=("parallel",)),
    )(input_points, feats, w1p8, w1fp, w2p, b2f, w3lp, b3f, w3q8, w4p, b4f)

    if Q_pad != Q:
        return out_pad[:, :, :Q]
    return out_pad


# speedup vs baseline: 1.2649x; 1.2604x over previous
import jax
import jax.numpy as jnp
from jax.experimental import pallas as pl
from jax.experimental.pallas import tpu as pltpu

_LANE = 128
_SUBLANE = 8
_ROW_TILE = 2048


def _round_up(x, m):
    return (x + m - 1) // m * m


def _pad2(w, rows, cols, dtype):
    return jnp.pad(w, ((0, rows - w.shape[0]), (0, cols - w.shape[1]))).astype(dtype)


def _enc_kernel(xt_ref, w1t_ref, w2t_ref, b2c_ref, w3l_ref, b3_ref,
                bias_ref, lat_ref):
    t = pl.program_id(1)
    nt = pl.num_programs(1)
    xt = xt_ref[0]
    ht = jnp.dot(w1t_ref[...], xt, preferred_element_type=jnp.float32)
    hb = jnp.maximum(ht.astype(jnp.bfloat16), 0)
    zt = jnp.dot(w2t_ref[...], hb, preferred_element_type=jnp.float32)
    zm = jnp.max(zt, axis=1, keepdims=True)

    @pl.when(t == 0)
    def _():
        lat_ref[...] = zm

    @pl.when(t > 0)
    def _():
        lat_ref[...] = jnp.maximum(lat_ref[...], zm)

    @pl.when(t == nt - 1)
    def _():
        lat = jnp.maximum(lat_ref[...] + b2c_ref[...], 0.0)
        lat_row = jnp.transpose(lat).astype(jnp.bfloat16)
        bias_ref[0] = (jnp.dot(lat_row, w3l_ref[...],
                               preferred_element_type=jnp.float32)
                       + b3_ref[...])


def _dec_kernel(x8_ref, w3qb_ref, w4_ref, b4_ref, o_ref):
    x8 = x8_ref[0]
    h = jax.lax.dot_general(x8, w3qb_ref[0],
                            (((0,), (0,)), ((), ())),
                            preferred_element_type=jnp.float32)
    hb = jnp.maximum(h.astype(jnp.bfloat16), 0)
    o_ref[0] = (jnp.dot(hb, w4_ref[...], preferred_element_type=jnp.float32)
                + b4_ref[...])


def kernel(input_points, input_features, w1p, w1f, b1, w2, b2,
           w3q, w3l, b3, w4, b4):
    B, N, _ = input_points.shape
    D = input_features.shape[-1]
    H = w1p.shape[-1]
    L = w2.shape[-1]
    Q = w4.shape[-1]

    C = 4 + D
    C_pad = _round_up(C, 2 * _SUBLANE)
    H_pad = _round_up(H, _LANE)
    L_pad = _round_up(L, _LANE)
    Q_pad = _round_up(Q, _LANE)
    TN = min(_ROW_TILE, _round_up(N, _LANE))
    N_pad = _round_up(N, TN)

    x = jnp.concatenate(
        [input_points, jnp.ones((B, N, 1), jnp.float32), input_features],
        axis=-1).astype(jnp.bfloat16)
    xt = jnp.transpose(x, (0, 2, 1))
    xt = jnp.pad(xt, ((0, 0), (0, C_pad - C), (0, 0)))
    if N_pad != N:
        xt = jnp.pad(xt, ((0, 0), (0, 0), (0, N_pad - N)), mode="edge")

    bf16 = jnp.bfloat16
    w1 = jnp.concatenate([w1p, b1, w1f], axis=0)
    w1t = _pad2(jnp.transpose(w1), H_pad, C_pad, bf16)
    w2t = _pad2(jnp.transpose(w2), L_pad, H_pad, bf16)
    b2c = _pad2(jnp.transpose(b2), L_pad, 1, jnp.float32)
    w3lp = _pad2(w3l, L_pad, H_pad, bf16)
    b3f = _pad2(b3, 1, H_pad, jnp.float32)
    w4p = _pad2(w4, H_pad, Q_pad, bf16)
    b4f = _pad2(b4, 1, Q_pad, jnp.float32)

    full = lambda shape: pl.BlockSpec(shape, lambda b, t: (0,) * len(shape))
    bias = pl.pallas_call(
        _enc_kernel,
        out_shape=jax.ShapeDtypeStruct((B, 1, H_pad), jnp.float32),
        grid=(B, N_pad // TN),
        in_specs=[
            pl.BlockSpec((1, C_pad, TN), lambda b, t: (b, 0, t)),
            full((H_pad, C_pad)),
            full((L_pad, H_pad)),
            full((L_pad, 1)),
            full((L_pad, H_pad)),
            full((1, H_pad)),
        ],
        out_specs=pl.BlockSpec((1, 1, H_pad), lambda b, t: (b, 0, 0)),
        scratch_shapes=[pltpu.VMEM((L_pad, 1), jnp.float32)],
        compiler_params=pltpu.CompilerParams(
            dimension_semantics=("parallel", "arbitrary")),
    )(xt, w1t, w2t, b2c, w3lp, b3f)

    w3q8 = jnp.pad(w3q, ((0, 0), (0, H_pad - H)))
    w3qb = jnp.concatenate(
        [jnp.broadcast_to(w3q8[None], (B, 3, H_pad)),
         bias,
         jnp.zeros((B, 4, H_pad), jnp.float32)], axis=1).astype(bf16)

    out_pad = pl.pallas_call(
        _dec_kernel,
        out_shape=jax.ShapeDtypeStruct((B, N_pad, Q_pad), jnp.float32),
        grid=(B, N_pad // TN),
        in_specs=[
            pl.BlockSpec((1, 8, TN), lambda b, t: (b, 0, t)),
            pl.BlockSpec((1, 8, H_pad), lambda b, t: (b, 0, 0)),
            full((H_pad, Q_pad)),
            full((1, Q_pad)),
        ],
        out_specs=pl.BlockSpec((1, TN, Q_pad), lambda b, t: (b, t, 0)),
        compiler_params=pltpu.CompilerParams(
            dimension_semantics=("parallel", "parallel")),
    )(xt, w3qb, w4p, b4f)

    if N_pad != N or Q_pad != Q:
        return out_pad[:, :N, :Q]
    return out_pad


# speedup vs baseline: 1.4524x; 1.1483x over previous
import jax
import jax.numpy as jnp
from jax.experimental import pallas as pl
from jax.experimental.pallas import tpu as pltpu

_LANE = 128
_SUBLANE = 8
_ROW_TILE = 8192


def _round_up(x, m):
    return (x + m - 1) // m * m


def _pad2(w, rows, cols, dtype):
    return jnp.pad(w, ((0, rows - w.shape[0]), (0, cols - w.shape[1]))).astype(dtype)


def _enc_kernel(xt_ref, w1t_ref, w2t_ref, b2c_ref, w3l_ref, b3_ref,
                bias_ref, lat_ref):
    t = pl.program_id(1)
    nt = pl.num_programs(1)
    xt = xt_ref[0]
    ht = jnp.dot(w1t_ref[...], xt, preferred_element_type=jnp.float32)
    hb = jnp.maximum(ht.astype(jnp.bfloat16), 0)
    zt = jnp.dot(w2t_ref[...], hb, preferred_element_type=jnp.float32)
    zm = jnp.max(zt, axis=1, keepdims=True)

    @pl.when(t == 0)
    def _():
        lat_ref[...] = zm

    @pl.when(t > 0)
    def _():
        lat_ref[...] = jnp.maximum(lat_ref[...], zm)

    @pl.when(t == nt - 1)
    def _():
        lat = jnp.maximum(lat_ref[...] + b2c_ref[...], 0.0)
        lat_row = jnp.transpose(lat).astype(jnp.bfloat16)
        bias_ref[0] = (jnp.dot(lat_row, w3l_ref[...],
                               preferred_element_type=jnp.float32)
                       + b3_ref[...])


def _dec_kernel(x8_ref, w3qb_ref, w4_ref, b4_ref, o_ref):
    x8 = x8_ref[0]
    h = jax.lax.dot_general(x8, w3qb_ref[0],
                            (((0,), (0,)), ((), ())),
                            preferred_element_type=jnp.float32)
    hb = jnp.maximum(h.astype(jnp.bfloat16), 0)
    o_ref[0] = (jnp.dot(hb, w4_ref[...], preferred_element_type=jnp.float32)
                + b4_ref[...])


def kernel(input_points, input_features, w1p, w1f, b1, w2, b2,
           w3q, w3l, b3, w4, b4):
    B, N, _ = input_points.shape
    D = input_features.shape[-1]
    H = w1p.shape[-1]
    L = w2.shape[-1]
    Q = w4.shape[-1]

    C = 4 + D
    C_pad = _round_up(C, 2 * _SUBLANE)
    H_pad = _round_up(H, _LANE)
    L_pad = _round_up(L, _LANE)
    Q_pad = _round_up(Q, _LANE)
    TN = min(_ROW_TILE, _round_up(N, _LANE))
    N_pad = _round_up(N, TN)

    x = jnp.concatenate(
        [input_points, jnp.ones((B, N, 1), jnp.float32), input_features],
        axis=-1).astype(jnp.bfloat16)
    xt = jnp.transpose(x, (0, 2, 1))
    xt = jnp.pad(xt, ((0, 0), (0, C_pad - C), (0, 0)))
    if N_pad != N:
        xt = jnp.pad(xt, ((0, 0), (0, 0), (0, N_pad - N)), mode="edge")

    bf16 = jnp.bfloat16
    w1 = jnp.concatenate([w1p, b1, w1f], axis=0)
    w1t = _pad2(jnp.transpose(w1), H_pad, C_pad, bf16)
    w2t = _pad2(jnp.transpose(w2), L_pad, H_pad, bf16)
    b2c = _pad2(jnp.transpose(b2), L_pad, 1, jnp.float32)
    w3lp = _pad2(w3l, L_pad, H_pad, bf16)
    b3f = _pad2(b3, 1, H_pad, jnp.float32)
    w4p = _pad2(w4, H_pad, Q_pad, bf16)
    b4f = _pad2(b4, 1, Q_pad, jnp.float32)

    full = lambda shape: pl.BlockSpec(shape, lambda b, t: (0,) * len(shape))
    bias = pl.pallas_call(
        _enc_kernel,
        out_shape=jax.ShapeDtypeStruct((B, 1, H_pad), jnp.float32),
        grid=(B, N_pad // TN),
        in_specs=[
            pl.BlockSpec((1, C_pad, TN), lambda b, t: (b, 0, t)),
            full((H_pad, C_pad)),
            full((L_pad, H_pad)),
            full((L_pad, 1)),
            full((L_pad, H_pad)),
            full((1, H_pad)),
        ],
        out_specs=pl.BlockSpec((1, 1, H_pad), lambda b, t: (b, 0, 0)),
        scratch_shapes=[pltpu.VMEM((L_pad, 1), jnp.float32)],
        compiler_params=pltpu.CompilerParams(
            dimension_semantics=("parallel", "arbitrary")),
    )(xt, w1t, w2t, b2c, w3lp, b3f)

    w3q8 = jnp.pad(w3q, ((0, 0), (0, H_pad - H)))
    w3qb = jnp.concatenate(
        [jnp.broadcast_to(w3q8[None], (B, 3, H_pad)),
         bias,
         jnp.zeros((B, 4, H_pad), jnp.float32)], axis=1).astype(bf16)

    out_pad = pl.pallas_call(
        _dec_kernel,
        out_shape=jax.ShapeDtypeStruct((B, N_pad, Q_pad), jnp.float32),
        grid=(B, N_pad // TN),
        in_specs=[
            pl.BlockSpec((1, 8, TN), lambda b, t: (b, 0, t)),
            pl.BlockSpec((1, 8, H_pad), lambda b, t: (b, 0, 0)),
            full((H_pad, Q_pad)),
            full((1, Q_pad)),
        ],
        out_specs=pl.BlockSpec((1, TN, Q_pad), lambda b, t: (b, t, 0)),
        compiler_params=pltpu.CompilerParams(
            dimension_semantics=("parallel", "parallel")),
    )(xt, w3qb, w4p, b4f)

    if N_pad != N or Q_pad != Q:
        return out_pad[:, :N, :Q]
    return out_pad


# speedup vs baseline: 1.5332x; 1.0557x over previous
import jax
import jax.numpy as jnp
from jax.experimental import pallas as pl
from jax.experimental.pallas import tpu as pltpu

_LANE = 128
_SUBLANE = 8
_ROW_TILE = 8192


def _round_up(x, m):
    return (x + m - 1) // m * m


def _pad2(w, rows, cols, dtype):
    return jnp.pad(w, ((0, rows - w.shape[0]), (0, cols - w.shape[1]))).astype(dtype)


def _enc_kernel(xt_ref, w1t_ref, w2t_ref, b2c_ref, w3l_ref, b3_ref,
                bias_ref, lat_ref):
    t = pl.program_id(1)
    nt = pl.num_programs(1)
    xt = xt_ref[0]
    ht = jnp.dot(w1t_ref[...], xt, preferred_element_type=jnp.float32)
    hb = jnp.maximum(ht.astype(jnp.bfloat16), 0)
    zt = jnp.dot(w2t_ref[...], hb, preferred_element_type=jnp.float32)
    zm = jnp.max(zt, axis=1, keepdims=True)

    @pl.when(t == 0)
    def _():
        lat_ref[...] = zm

    @pl.when(t > 0)
    def _():
        lat_ref[...] = jnp.maximum(lat_ref[...], zm)

    @pl.when(t == nt - 1)
    def _():
        lat = jnp.maximum(lat_ref[...] + b2c_ref[...], 0.0)
        lat_row = jnp.transpose(lat).astype(jnp.bfloat16)
        bias_ref[0] = (jnp.dot(lat_row, w3l_ref[...],
                               preferred_element_type=jnp.float32)
                       + b3_ref[...])


def _dec_kernel(x8_ref, w3qb_ref, w4t_ref, b4c_ref, o_ref):
    x8 = x8_ref[0]
    ht = jnp.dot(w3qb_ref[0], x8,
                 preferred_element_type=jnp.float32)
    hb = jnp.maximum(ht.astype(jnp.bfloat16), 0)
    ot = (jnp.dot(w4t_ref[...], hb, preferred_element_type=jnp.float32)
          + b4c_ref[...])
    o_ref[0] = jnp.transpose(ot)


def kernel(input_points, input_features, w1p, w1f, b1, w2, b2,
           w3q, w3l, b3, w4, b4):
    B, N, _ = input_points.shape
    D = input_features.shape[-1]
    H = w1p.shape[-1]
    L = w2.shape[-1]
    Q = w4.shape[-1]

    C = 4 + D
    C_pad = _round_up(C, 2 * _SUBLANE)
    H_pad = _round_up(H, _LANE)
    L_pad = _round_up(L, _LANE)
    Q_pad = _round_up(Q, _LANE)
    TN = min(_ROW_TILE, _round_up(N, _LANE))
    N_pad = _round_up(N, TN)

    x = jnp.concatenate(
        [input_points, jnp.ones((B, N, 1), jnp.float32), input_features],
        axis=-1).astype(jnp.bfloat16)
    xt = jnp.transpose(x, (0, 2, 1))
    xt = jnp.pad(xt, ((0, 0), (0, C_pad - C), (0, 0)))
    if N_pad != N:
        xt = jnp.pad(xt, ((0, 0), (0, 0), (0, N_pad - N)), mode="edge")

    bf16 = jnp.bfloat16
    w1 = jnp.concatenate([w1p, b1, w1f], axis=0)
    w1t = _pad2(jnp.transpose(w1), H_pad, C_pad, bf16)
    w2t = _pad2(jnp.transpose(w2), L_pad, H_pad, bf16)
    b2c = _pad2(jnp.transpose(b2), L_pad, 1, jnp.float32)
    w3lp = _pad2(w3l, L_pad, H_pad, bf16)
    b3f = _pad2(b3, 1, H_pad, jnp.float32)
    w4t = _pad2(jnp.transpose(w4), Q_pad, H_pad, bf16)
    b4c = _pad2(jnp.transpose(b4), Q_pad, 1, jnp.float32)

    full = lambda shape: pl.BlockSpec(shape, lambda b, t: (0,) * len(shape))
    bias = pl.pallas_call(
        _enc_kernel,
        out_shape=jax.ShapeDtypeStruct((B, 1, H_pad), jnp.float32),
        grid=(B, N_pad // TN),
        in_specs=[
            pl.BlockSpec((1, C_pad, TN), lambda b, t: (b, 0, t)),
            full((H_pad, C_pad)),
            full((L_pad, H_pad)),
            full((L_pad, 1)),
            full((L_pad, H_pad)),
            full((1, H_pad)),
        ],
        out_specs=pl.BlockSpec((1, 1, H_pad), lambda b, t: (b, 0, 0)),
        scratch_shapes=[pltpu.VMEM((L_pad, 1), jnp.float32)],
        compiler_params=pltpu.CompilerParams(
            dimension_semantics=("parallel", "arbitrary")),
    )(xt, w1t, w2t, b2c, w3lp, b3f)

    w3qt = jnp.pad(jnp.transpose(w3q), ((0, H_pad - H), (0, 0)))
    w3qb = jnp.concatenate(
        [jnp.broadcast_to(w3qt[None], (B, H_pad, 3)),
         jnp.transpose(bias, (0, 2, 1)),
         jnp.zeros((B, H_pad, 4), jnp.float32)], axis=2).astype(bf16)

    out_pad = pl.pallas_call(
        _dec_kernel,
        out_shape=jax.ShapeDtypeStruct((B, N_pad, Q_pad), jnp.float32),
        grid=(B, N_pad // TN),
        in_specs=[
            pl.BlockSpec((1, 8, TN), lambda b, t: (b, 0, t)),
            pl.BlockSpec((1, H_pad, 8), lambda b, t: (b, 0, 0)),
            full((Q_pad, H_pad)),
            full((Q_pad, 1)),
        ],
        out_specs=pl.BlockSpec((1, TN, Q_pad), lambda b, t: (b, t, 0)),
        compiler_params=pltpu.CompilerParams(
            dimension_semantics=("parallel", "parallel")),
    )(xt, w3qb, w4t, b4c)

    if N_pad != N or Q_pad != Q:
        return out_pad[:, :N, :Q]
    return out_pad


# speedup vs baseline: 1.5492x; 1.0104x over previous
import jax
import jax.numpy as jnp
from jax.experimental import pallas as pl
from jax.experimental.pallas import tpu as pltpu

_LANE = 128
_SUBLANE = 8
_ROW_TILE = 8192


def _round_up(x, m):
    return (x + m - 1) // m * m


def _pad2(w, rows, cols, dtype):
    return jnp.pad(w, ((0, rows - w.shape[0]), (0, cols - w.shape[1]))).astype(dtype)


def _enc_kernel(xt_ref, w1t_ref, w2t_ref, b2c_ref, w3l_ref, b3_ref,
                bias_ref, lat_ref):
    t = pl.program_id(1)
    nt = pl.num_programs(1)
    xt = xt_ref[0]
    ht = jnp.dot(w1t_ref[...], xt, preferred_element_type=jnp.float32)
    hb = jnp.maximum(ht.astype(jnp.bfloat16), 0)
    zt = jnp.dot(w2t_ref[...], hb, preferred_element_type=jnp.float32)
    zm = jnp.max(zt, axis=1, keepdims=True)

    @pl.when(t == 0)
    def _():
        lat_ref[...] = zm

    @pl.when(t > 0)
    def _():
        lat_ref[...] = jnp.maximum(lat_ref[...], zm)

    @pl.when(t == nt - 1)
    def _():
        lat = jnp.maximum(lat_ref[...] + b2c_ref[...], 0.0)
        lat_row = jnp.transpose(lat).astype(jnp.bfloat16)
        bias_ref[0] = (jnp.dot(lat_row, w3l_ref[...],
                               preferred_element_type=jnp.float32)
                       + b3_ref[...])


def _dec_kernel(x8_ref, w3qb_ref, w4t_ref, b4c_ref, o_ref):
    x8 = x8_ref[0]
    ht = jnp.dot(w3qb_ref[0], x8,
                 preferred_element_type=jnp.float32)
    hb = jnp.maximum(ht.astype(jnp.bfloat16), 0)
    ot = (jnp.dot(w4t_ref[...], hb, preferred_element_type=jnp.float32)
          + b4c_ref[...])
    o_ref[0] = jnp.transpose(ot)


def kernel(input_points, input_features, w1p, w1f, b1, w2, b2,
           w3q, w3l, b3, w4, b4):
    B, N, _ = input_points.shape
    D = input_features.shape[-1]
    H = w1p.shape[-1]
    L = w2.shape[-1]
    Q = w4.shape[-1]

    C = 4 + D
    C_pad = _round_up(C, 2 * _SUBLANE)
    H_pad = _round_up(H, _LANE)
    L_pad = _round_up(L, _LANE)
    Q_pad = _round_up(Q, _LANE)
    TN = min(_ROW_TILE, _round_up(N, _LANE))
    N_pad = _round_up(N, TN)

    x = jnp.concatenate(
        [input_points, jnp.ones((B, N, 1), jnp.float32), input_features,
         jnp.zeros((B, N, C_pad - C), jnp.float32)],
        axis=-1).astype(jnp.bfloat16)
    xt = jnp.transpose(x, (0, 2, 1))
    if N_pad != N:
        xt = jnp.pad(xt, ((0, 0), (0, 0), (0, N_pad - N)), mode="edge")

    bf16 = jnp.bfloat16
    w1 = jnp.concatenate([w1p, b1, w1f], axis=0)
    w1t = _pad2(jnp.transpose(w1), H_pad, C_pad, bf16)
    w2t = _pad2(jnp.transpose(w2), L_pad, H_pad, bf16)
    b2c = _pad2(jnp.transpose(b2), L_pad, 1, jnp.float32)
    w3lp = _pad2(w3l, L_pad, H_pad, bf16)
    b3f = _pad2(b3, 1, H_pad, jnp.float32)
    w4t = _pad2(jnp.transpose(w4), Q_pad, H_pad, bf16)
    b4c = _pad2(jnp.transpose(b4), Q_pad, 1, jnp.float32)

    full = lambda shape: pl.BlockSpec(shape, lambda b, t: (0,) * len(shape))
    bias = pl.pallas_call(
        _enc_kernel,
        out_shape=jax.ShapeDtypeStruct((B, 1, H_pad), jnp.float32),
        grid=(B, N_pad // TN),
        in_specs=[
            pl.BlockSpec((1, C_pad, TN), lambda b, t: (b, 0, t)),
            full((H_pad, C_pad)),
            full((L_pad, H_pad)),
            full((L_pad, 1)),
            full((L_pad, H_pad)),
            full((1, H_pad)),
        ],
        out_specs=pl.BlockSpec((1, 1, H_pad), lambda b, t: (b, 0, 0)),
        scratch_shapes=[pltpu.VMEM((L_pad, 1), jnp.float32)],
        compiler_params=pltpu.CompilerParams(
            dimension_semantics=("parallel", "arbitrary")),
    )(xt, w1t, w2t, b2c, w3lp, b3f)

    w3qt = jnp.pad(jnp.transpose(w3q), ((0, H_pad - H), (0, 0)))
    w3qb = jnp.concatenate(
        [jnp.broadcast_to(w3qt[None], (B, H_pad, 3)),
         jnp.transpose(bias, (0, 2, 1)),
         jnp.zeros((B, H_pad, 4), jnp.float32)], axis=2).astype(bf16)

    out_pad = pl.pallas_call(
        _dec_kernel,
        out_shape=jax.ShapeDtypeStruct((B, N_pad, Q_pad), jnp.float32),
        grid=(B, N_pad // TN),
        in_specs=[
            pl.BlockSpec((1, 8, TN), lambda b, t: (b, 0, t)),
            pl.BlockSpec((1, H_pad, 8), lambda b, t: (b, 0, 0)),
            full((Q_pad, H_pad)),
            full((Q_pad, 1)),
        ],
        out_specs=pl.BlockSpec((1, TN, Q_pad), lambda b, t: (b, t, 0)),
        compiler_params=pltpu.CompilerParams(
            dimension_semantics=("parallel", "parallel")),
    )(xt, w3qb, w4t, b4c)

    if N_pad != N or Q_pad != Q:
        return out_pad[:, :N, :Q]
    return out_pad


# speedup vs baseline: 1.5501x; 1.0006x over previous
import jax
import jax.numpy as jnp
from jax.experimental import pallas as pl
from jax.experimental.pallas import tpu as pltpu

_LANE = 128
_SUBLANE = 8
_ROW_TILE = 8192


def _round_up(x, m):
    return (x + m - 1) // m * m


def _enc_kernel(xt_ref, wp_ref, bias_ref, lat_ref, dims):
    C_pad, H, L, Q = dims
    t = pl.program_id(1)
    nt = pl.num_programs(1)
    xt = xt_ref[0]
    w1n = wp_ref[0:C_pad]
    ht = jax.lax.dot_general(w1n, xt, (((0,), (0,)), ((), ())),
                             preferred_element_type=jnp.float32)
    hb = jnp.maximum(ht.astype(jnp.bfloat16), 0)
    w2t = wp_ref[C_pad:C_pad + L]
    zt = jnp.dot(w2t, hb, preferred_element_type=jnp.float32)
    zm = jnp.max(zt, axis=1, keepdims=True)

    @pl.when(t == 0)
    def _():
        lat_ref[...] = zm

    @pl.when(t > 0)
    def _():
        lat_ref[...] = jnp.maximum(lat_ref[...], zm)

    @pl.when(t == nt - 1)
    def _():
        r = C_pad + 2 * L + Q
        b2c = jnp.transpose(wp_ref[r + 1:r + 2, 0:L]).astype(jnp.float32)
        lat = jnp.maximum(lat_ref[...] + b2c, 0.0)
        lat_row = jnp.transpose(lat).astype(jnp.bfloat16)
        w3l = wp_ref[C_pad + L:C_pad + 2 * L]
        b3 = wp_ref[r:r + 1].astype(jnp.float32)
        bias_ref[0] = (jnp.dot(lat_row, w3l,
                               preferred_element_type=jnp.float32) + b3)


def _dec_kernel(x8_ref, w3qb_ref, wp_ref, o_ref, dims):
    C_pad, H, L, Q = dims
    x8 = x8_ref[0]
    ht = jnp.dot(w3qb_ref[0], x8,
                 preferred_element_type=jnp.float32)
    hb = jnp.maximum(ht.astype(jnp.bfloat16), 0)
    w4t = wp_ref[C_pad + 2 * L:C_pad + 2 * L + Q]
    r = C_pad + 2 * L + Q
    b4c = jnp.transpose(wp_ref[r + 2:r + 3, 0:Q]).astype(jnp.float32)
    ot = (jnp.dot(w4t, hb, preferred_element_type=jnp.float32)
          + b4c)
    o_ref[0] = jnp.transpose(ot)


def kernel(input_points, input_features, w1p, w1f, b1, w2, b2,
           w3q, w3l, b3, w4, b4):
    B, N, _ = input_points.shape
    D = input_features.shape[-1]
    H = w1p.shape[-1]
    L = w2.shape[-1]
    Q = w4.shape[-1]

    C = 4 + D
    C_pad = _round_up(C, 2 * _SUBLANE)
    H_pad = _round_up(H, _LANE)
    TN = min(_ROW_TILE, _round_up(N, _LANE))
    N_pad = _round_up(N, TN)
    dims = (C_pad, H, L, Q)

    x = jnp.concatenate(
        [input_points, jnp.ones((B, N, 1), jnp.float32), input_features,
         jnp.zeros((B, N, C_pad - C), jnp.float32)],
        axis=-1).astype(jnp.bfloat16)
    xt = jnp.transpose(x, (0, 2, 1))
    if N_pad != N:
        xt = jnp.pad(xt, ((0, 0), (0, 0), (0, N_pad - N)), mode="edge")

    bf16 = jnp.bfloat16
    rows = C_pad + 2 * L + Q + 3
    wpack = jnp.concatenate([
        w1p, b1, w1f, jnp.zeros((C_pad - C, H), jnp.float32),
        jnp.transpose(w2),
        w3l,
        jnp.transpose(w4),
        b3,
        jnp.pad(b2, ((0, 0), (0, H - L))),
        jnp.pad(b4, ((0, 0), (0, H - Q))),
    ], axis=0).astype(bf16)
    R_pad = _round_up(rows, 2 * _SUBLANE)
    if R_pad != rows:
        wpack = jnp.pad(wpack, ((0, R_pad - rows), (0, 0)))

    full = lambda shape: pl.BlockSpec(shape, lambda b, t: (0,) * len(shape))
    import functools
    bias = pl.pallas_call(
        functools.partial(_enc_kernel, dims=dims),
        out_shape=jax.ShapeDtypeStruct((B, 1, H), jnp.float32),
        grid=(B, N_pad // TN),
        in_specs=[
            pl.BlockSpec((1, C_pad, TN), lambda b, t: (b, 0, t)),
            full((R_pad, H)),
        ],
        out_specs=pl.BlockSpec((1, 1, H), lambda b, t: (b, 0, 0)),
        scratch_shapes=[pltpu.VMEM((L, 1), jnp.float32)],
        compiler_params=pltpu.CompilerParams(
            dimension_semantics=("parallel", "arbitrary")),
    )(xt, wpack)

    w3qt = jnp.transpose(w3q)
    w3qb = jnp.concatenate(
        [jnp.broadcast_to(w3qt[None], (B, H, 3)),
         jnp.transpose(bias, (0, 2, 1)),
         jnp.zeros((B, H, 4), jnp.float32)], axis=2).astype(bf16)

    out = pl.pallas_call(
        functools.partial(_dec_kernel, dims=dims),
        out_shape=jax.ShapeDtypeStruct((B, N_pad, Q), jnp.float32),
        grid=(B, N_pad // TN),
        in_specs=[
            pl.BlockSpec((1, 8, TN), lambda b, t: (b, 0, t)),
            pl.BlockSpec((1, H, 8), lambda b, t: (b, 0, 0)),
            full((R_pad, H)),
        ],
        out_specs=pl.BlockSpec((1, TN, Q), lambda b, t: (b, t, 0)),
        compiler_params=pltpu.CompilerParams(
            dimension_semantics=("parallel", "parallel")),
    )(xt, w3qb, wpack)

    if N_pad != N:
        return out[:, :N, :]
    return out
